# baseline (speedup 1.0000x reference)
import functools
import math

import jax
import jax.numpy as jnp
from jax import lax
from jax.experimental import pallas as pl
from jax.experimental.pallas import tpu as pltpu
from jax.experimental.pallas import tpu_sc as plsc

LMAX = 2
H = 8
DH = 32
C = 128
K = 32
OUT = H * DH

NC = 2
NS = 16
NW = NC * NS

CH_NODES = 2
CH_EDGES = CH_NODES * K

BN = 400


def _tc_body(node_ref, xe_ref, Wq_ref, bq_ref, Wk_ref, bk_ref,
             W1_ref, b1_ref, lng_ref, lnb_ref, W2_ref, b2_ref,
             q_ref, k_ref, es_ref):
    x0 = node_ref[:, 0, :]

    def proj(W_ref, b_ref):
        out = jnp.dot(x0, W_ref[0], preferred_element_type=jnp.float32) + b_ref[0]
        for l in (1, 2):
            sq = None
            for m in range(l * l, (l + 1) * (l + 1)):
                y = jnp.dot(node_ref[:, m, :], W_ref[l],
                            preferred_element_type=jnp.float32)
                sq = y * y if sq is None else sq + y * y
            out = out + jnp.sqrt(sq + 1e-12)
        return out

    q_ref[...] = proj(Wq_ref, bq_ref) * (1.0 / math.sqrt(float(DH)))
    k_ref[...] = proj(Wk_ref, bk_ref)

    xe = xe_ref[...]
    h = jnp.dot(xe, W1_ref[...], preferred_element_type=jnp.float32) + b1_ref[0]
    ones = jnp.ones((32, 1), jnp.float32)
    mu = jnp.dot(h, ones, preferred_element_type=jnp.float32) * (1.0 / 32.0)
    msq = jnp.dot(h * h, ones, preferred_element_type=jnp.float32) * (1.0 / 32.0)
    var = msq - mu * mu
    hn = (h - mu) * lax.rsqrt(var + 1e-5) * lng_ref[0] + lnb_ref[0]
    hs = hn / (1.0 + jnp.exp(-hn))
    es_ref[...] = jnp.dot(hs, W2_ref[...], preferred_element_type=jnp.float32) + b2_ref[0]


def _tc_stage(node, xe2, Wq, bq, Wk, bk, W1, b1, lng, lnb, W2, b2, *, interpret=False):
    n = node.shape[0]
    grid = n // BN
    full = lambda shape: pl.BlockSpec(shape, lambda i: (0,) * len(shape))
    return pl.pallas_call(
        _tc_body,
        grid=(grid,),
        in_specs=[
            pl.BlockSpec((BN, (LMAX + 1) ** 2, C), lambda i: (i, 0, 0)),
            pl.BlockSpec((BN * K, 16), lambda i: (i, 0)),
            full((LMAX + 1, C, OUT)),
            full((1, OUT)),
            full((LMAX + 1, C, OUT)),
            full((1, OUT)),
            full((16, 32)),
            full((1, 32)),
            full((1, 32)),
            full((1, 32)),
            full((32, H)),
            full((1, H)),
        ],
        out_specs=[
            pl.BlockSpec((BN, OUT), lambda i: (i, 0)),
            pl.BlockSpec((BN, OUT), lambda i: (i, 0)),
            pl.BlockSpec((BN * K, H), lambda i: (i, 0)),
        ],
        out_shape=[
            jax.ShapeDtypeStruct((n, OUT), jnp.float32),
            jax.ShapeDtypeStruct((n, OUT), jnp.float32),
            jax.ShapeDtypeStruct((n * K, H), jnp.float32),
        ],
        interpret=interpret,
    )(node, xe2, Wq, bq, Wk, bk, W1, b1, lng, lnb, W2, b2)


def _make_sc_kernel(n_nodes, interpret=False):
    n_chunks = n_nodes // CH_NODES
    base_ch = n_chunks // NW
    rem = n_chunks - base_ch * NW
    scale = 1.0 / math.sqrt(float(DH))
    mesh = plsc.VectorSubcoreMesh(core_axis_name="c", subcore_axis_name="s")

    @functools.partial(
        pl.kernel,
        out_type=jax.ShapeDtypeStruct((n_nodes * K, H), jnp.float32),
        mesh=mesh,
        scratch_types=[
            pltpu.VMEM((base_ch * CH_EDGES,), jnp.int32),
            pltpu.VMEM((CH_EDGES,), jnp.int32),
            pltpu.VMEM((CH_EDGES, OUT), jnp.float32),
            pltpu.VMEM((CH_EDGES, OUT), jnp.float32),
            pltpu.VMEM((CH_NODES, OUT), jnp.float32),
            pltpu.VMEM((CH_NODES, OUT), jnp.float32),
            pltpu.VMEM((CH_EDGES, H), jnp.float32),
            pltpu.VMEM((CH_EDGES, H), jnp.float32),
            pltpu.VMEM((CH_EDGES, H), jnp.float32),
            pltpu.VMEM((CH_EDGES, H), jnp.float32),
            pltpu.SemaphoreType.DMA,
            pltpu.SemaphoreType.DMA,
            pltpu.SemaphoreType.DMA,
            pltpu.SemaphoreType.DMA,
        ],
        compiler_params=pltpu.CompilerParams(needs_layout_passes=False),
        interpret=interpret,
    )
    def sc_kernel(q_hbm, k_hbm, idx_hbm, es_hbm, out_hbm,
                  idx_all, idx_x, krows0, krows1, qc0, qc1, esc0, esc1,
                  outc0, outc1, in_sem0, in_sem1, out_sem0, out_sem1):
        cid = lax.axis_index("c")
        sid = lax.axis_index("s")
        wid = sid * NC + cid
        start = wid * base_ch
        iota = lax.iota(jnp.int32, 16)
        bufs = ((krows0, qc0, esc0, outc0, in_sem0, out_sem0),
                (krows1, qc1, esc1, outc1, in_sem1, out_sem1))

        def in_descs(c, b):
            krows, qc, esc, _, in_sem, _ = bufs[b]
            nbase = (start + c) * CH_NODES
            idx_ref = idx_all.at[pl.ds(c * CH_EDGES, CH_EDGES)]
            return (
                pltpu.make_async_copy(k_hbm.at[idx_ref], krows, in_sem),
                pltpu.make_async_copy(q_hbm.at[pl.ds(nbase, CH_NODES)], qc, in_sem),
                pltpu.make_async_copy(
                    es_hbm.at[pl.ds(nbase * K, CH_EDGES)], esc, in_sem),
            )

        def out_desc(c, b):
            outc, out_sem = bufs[b][3], bufs[b][5]
            ebase = (start + c) * CH_EDGES
            return pltpu.make_async_copy(outc, out_hbm.at[pl.ds(ebase, CH_EDGES)],
                                         out_sem)

        def compute(krows, qc, esc, outc):
            for nn in range(CH_NODES):
                accs = []
                rows0 = iota + (nn * K)
                rows1 = rows0 + 16
                nnv = jnp.full((16,), nn, jnp.int32)
                for h in range(H):

                    def dbody(d2, accs_, h=h, nnv=nnv, rows0=rows0, rows1=rows1):
                        a0, a1 = accs_
                        colv = h * DH + ((d2 + iota) & (DH - 1))
                        qvv = plsc.load_gather(qc, [nnv, colv])
                        a0 = a0 + plsc.load_gather(krows, [rows0, colv]) * qvv
                        a1 = a1 + plsc.load_gather(krows, [rows1, colv]) * qvv
                        return (a0, a1)

                    z = jnp.zeros((16,), jnp.float32)
                    a0, a1 = lax.fori_loop(0, DH, dbody, (z, z), unroll=16)
                    accs.append((rows0, h, a0))
                    accs.append((rows1, h, a1))
                GB = 8
                for g0 in range(0, len(accs), GB):
                    grp = accs[g0:g0 + GB]
                    ss = [acc * plsc.load_gather(esc, [rows, jnp.full((16,), h, jnp.int32)])
                          for rows, h, acc in grp]
                    ts = [jnp.exp(-s) for s in ss]
                    sigs = [1.0 / (1.0 + t) for t in ts]
                    for (rows, h, _), s, sig in zip(grp, ss, sigs):
                        alpha = s * (0.2 + 0.8 * sig)
                        plsc.store_scatter(
                            outc, [rows, jnp.full((16,), h, jnp.int32)], alpha)

        pltpu.sync_copy(idx_hbm.at[pl.ds(start * CH_EDGES, base_ch * CH_EDGES)],
                        idx_all)
        for b in (0, 1):
            for d in in_descs(b, b):
                d.start()

        @pl.loop(0, base_ch, step=2)
        def _pair(ci):
            for b in (0, 1):
                c = ci + b
                krows, qc, esc, outc, _, _ = bufs[b]

                @pl.when(c >= 2)
                def _drain_out():
                    out_desc(c - 2, b).wait()

                for d in in_descs(c, b):
                    d.wait()
                compute(krows, qc, esc, outc)
                out_desc(c, b).start()

                @pl.when(c + 2 < base_ch)
                def _prefetch():
                    for d in in_descs(c + 2, b):
                        d.start()

        out_desc(base_ch - 2, 0).wait()
        out_desc(base_ch - 1, 1).wait()

        @pl.when(wid < rem)
        def _extra():
            g = NW * base_ch + wid
            nbase = g * CH_NODES
            ebase = nbase * K
            pltpu.sync_copy(idx_hbm.at[pl.ds(ebase, CH_EDGES)], idx_x)
            pltpu.async_copy(k_hbm.at[idx_x], krows0, in_sem0).wait()
            pltpu.sync_copy(q_hbm.at[pl.ds(nbase, CH_NODES)], qc0)
            pltpu.sync_copy(es_hbm.at[pl.ds(ebase, CH_EDGES)], esc0)
            compute(krows0, qc0, esc0, outc0)
            pltpu.sync_copy(outc0, out_hbm.at[pl.ds(ebase, CH_EDGES)])

    return sc_kernel


def kernel(x_edge, node_irreps_input, f_sparse_idx_node, Wq, bq, Wk, bk,
           W1, b1, ln_g, ln_b, W2, b2):
    n = node_irreps_input.shape[0]
    xe2 = x_edge.reshape(n * K, 16)
    idx_flat = f_sparse_idx_node.astype(jnp.int32).reshape(n * K)
    qtab, ktab, es2 = _tc_stage(
        node_irreps_input, xe2, Wq, bq.reshape(1, OUT), Wk, bk.reshape(1, OUT),
        W1, b1.reshape(1, 32), ln_g.reshape(1, 32), ln_b.reshape(1, 32),
        W2, b2.reshape(1, H))
    alpha = _make_sc_kernel(n)(qtab, ktab, idx_flat, es2)
    return alpha.reshape(n, K, H)


# speedup vs baseline: 1.1230x; 1.1230x over previous
import functools
import math

import jax
import jax.numpy as jnp
from jax import lax
from jax.experimental import pallas as pl
from jax.experimental.pallas import tpu as pltpu
from jax.experimental.pallas import tpu_sc as plsc

LMAX = 2
H = 8
DH = 32
C = 128
K = 32
OUT = H * DH

NC = 2
NS = 16
NW = NC * NS

CH_NODES = 2
CH_EDGES = CH_NODES * K

BN = 400


def _tc_body(node_ref, xe_ref, Wq_ref, bq_ref, Wk_ref, bk_ref,
             W1bd_ref, b1bd_ref, mred_ref, mbc_ref, lngbd_ref, lnbbd_ref,
             W2bd_ref, b2bd_ref, qes_ref, k_ref):
    x0 = node_ref[:, 0, :]

    def proj(W_ref, b_ref):
        out = jnp.dot(x0, W_ref[0], preferred_element_type=jnp.float32) + b_ref[0]
        for l in (1, 2):
            sq = None
            for m in range(l * l, (l + 1) * (l + 1)):
                y = jnp.dot(node_ref[:, m, :], W_ref[l],
                            preferred_element_type=jnp.float32)
                sq = y * y if sq is None else sq + y * y
            out = out + jnp.sqrt(sq + 1e-12)
        return out

    qes_ref[:, :OUT] = proj(Wq_ref, bq_ref) * (1.0 / math.sqrt(float(DH)))
    k_ref[...] = proj(Wk_ref, bk_ref)

    xe = xe_ref[...]
    h = jnp.dot(xe, W1bd_ref[...], preferred_element_type=jnp.float32) + b1bd_ref[0]
    mu = jnp.dot(jnp.dot(h, mred_ref[...], preferred_element_type=jnp.float32),
                 mbc_ref[...], preferred_element_type=jnp.float32)
    d = h - mu
    var = jnp.dot(jnp.dot(d * d, mred_ref[...], preferred_element_type=jnp.float32),
                  mbc_ref[...], preferred_element_type=jnp.float32)
    hn = d * lax.rsqrt(var + 1e-5) * lngbd_ref[0] + lnbbd_ref[0]
    hs = hn / (1.0 + jnp.exp(-hn))
    qes_ref[:, OUT:] = jnp.dot(hs, W2bd_ref[...],
                               preferred_element_type=jnp.float32) + b2bd_ref[0]


def _tc_stage(node, xe2, Wq, bq, Wk, bk, W1bd, b1bd, mred, mbc, lngbd, lnbbd,
              W2bd, b2bd, *, interpret=False):
    n = node.shape[0]
    grid = n // BN
    full = lambda shape: pl.BlockSpec(shape, lambda i: (0,) * len(shape))
    return pl.pallas_call(
        _tc_body,
        grid=(grid,),
        in_specs=[
            pl.BlockSpec((BN, (LMAX + 1) ** 2, C), lambda i: (i, 0, 0)),
            pl.BlockSpec((BN, K * 16), lambda i: (i, 0)),
            full((LMAX + 1, C, OUT)),
            full((1, OUT)),
            full((LMAX + 1, C, OUT)),
            full((1, OUT)),
            full((K * 16, K * 32)),
            full((1, K * 32)),
            full((K * 32, K)),
            full((K, K * 32)),
            full((1, K * 32)),
            full((1, K * 32)),
            full((K * 32, K * H)),
            full((1, K * H)),
        ],
        out_specs=[
            pl.BlockSpec((BN, OUT + K * H), lambda i: (i, 0)),
            pl.BlockSpec((BN, OUT), lambda i: (i, 0)),
        ],
        out_shape=[
            jax.ShapeDtypeStruct((n, OUT + K * H), jnp.float32),
            jax.ShapeDtypeStruct((n, OUT), jnp.float32),
        ],
        interpret=interpret,
    )(node, xe2, Wq, bq, Wk, bk, W1bd, b1bd, mred, mbc, lngbd, lnbbd,
      W2bd, b2bd)


def _make_sc_kernel(n_nodes, interpret=False):
    n_chunks = n_nodes // CH_NODES
    base_ch = n_chunks // NW
    rem = n_chunks - base_ch * NW
    scale = 1.0 / math.sqrt(float(DH))
    mesh = plsc.VectorSubcoreMesh(core_axis_name="c", subcore_axis_name="s")

    @functools.partial(
        pl.kernel,
        out_type=jax.ShapeDtypeStruct((n_nodes * K, H), jnp.float32),
        mesh=mesh,
        scratch_types=[
            pltpu.VMEM((base_ch * CH_EDGES,), jnp.int32),
            pltpu.VMEM((CH_EDGES,), jnp.int32),
            pltpu.VMEM((CH_EDGES, OUT), jnp.float32),
            pltpu.VMEM((CH_EDGES, OUT), jnp.float32),
            pltpu.VMEM((CH_NODES, OUT + K * H), jnp.float32),
            pltpu.VMEM((CH_NODES, OUT + K * H), jnp.float32),
            pltpu.VMEM((CH_EDGES, H), jnp.float32),
            pltpu.VMEM((CH_EDGES, H), jnp.float32),
            pltpu.SemaphoreType.DMA,
            pltpu.SemaphoreType.DMA,
            pltpu.SemaphoreType.DMA,
            pltpu.SemaphoreType.DMA,
        ],
        compiler_params=pltpu.CompilerParams(needs_layout_passes=False),
        interpret=interpret,
    )
    def sc_kernel(qes_hbm, k_hbm, idx_hbm, out_hbm,
                  idx_all, idx_x, krows0, krows1, qc0, qc1,
                  outc0, outc1, in_sem0, in_sem1, out_sem0, out_sem1):
        cid = lax.axis_index("c")
        sid = lax.axis_index("s")
        wid = sid * NC + cid
        start = wid * base_ch
        iota = lax.iota(jnp.int32, 16)
        bufs = ((krows0, qc0, outc0, in_sem0, out_sem0),
                (krows1, qc1, outc1, in_sem1, out_sem1))

        def in_descs(c, b):
            krows, qc, _, in_sem, _ = bufs[b]
            nbase = (start + c) * CH_NODES
            idx_ref = idx_all.at[pl.ds(c * CH_EDGES, CH_EDGES)]
            return (
                pltpu.make_async_copy(k_hbm.at[idx_ref], krows, in_sem),
                pltpu.make_async_copy(qes_hbm.at[pl.ds(nbase, CH_NODES)], qc, in_sem),
            )

        def out_desc(c, b):
            outc, out_sem = bufs[b][2], bufs[b][4]
            ebase = (start + c) * CH_EDGES
            return pltpu.make_async_copy(outc, out_hbm.at[pl.ds(ebase, CH_EDGES)],
                                         out_sem)

        def compute(krows, qc, outc):
            for nn in range(CH_NODES):
                accs = []
                rows0 = iota + (nn * K)
                rows1 = rows0 + 16
                nnv = jnp.full((16,), nn, jnp.int32)
                for h in range(H):

                    def dbody(d2, accs_, h=h, nnv=nnv, rows0=rows0, rows1=rows1):
                        a0, a1 = accs_
                        colv = h * DH + ((d2 + iota) & (DH - 1))
                        qvv = plsc.load_gather(qc, [nnv, colv])
                        a0 = a0 + plsc.load_gather(krows, [rows0, colv]) * qvv
                        a1 = a1 + plsc.load_gather(krows, [rows1, colv]) * qvv
                        return (a0, a1)

                    z = jnp.zeros((16,), jnp.float32)
                    a0, a1 = lax.fori_loop(0, DH, dbody, (z, z), unroll=16)
                    accs.append((rows0, nnv, OUT + iota * H + h, h, a0))
                    accs.append((rows1, nnv, OUT + (iota + 16) * H + h, h, a1))
                GB = 8
                for g0 in range(0, len(accs), GB):
                    grp = accs[g0:g0 + GB]
                    ss = [acc * plsc.load_gather(qc, [nv, ecol])
                          for rows, nv, ecol, h, acc in grp]
                    ts = [jnp.exp(-s) for s in ss]
                    sigs = [1.0 / (1.0 + t) for t in ts]
                    for (rows, nv, ecol, h, _), s, sig in zip(grp, ss, sigs):
                        alpha = s * (0.2 + 0.8 * sig)
                        plsc.store_scatter(
                            outc, [rows, jnp.full((16,), h, jnp.int32)], alpha)

        pltpu.sync_copy(idx_hbm.at[pl.ds(start * CH_EDGES, base_ch * CH_EDGES)],
                        idx_all)
        for b in (0, 1):
            for d in in_descs(b, b):
                d.start()

        @pl.loop(0, base_ch, step=2)
        def _pair(ci):
            for b in (0, 1):
                c = ci + b
                krows, qc, outc, _, _ = bufs[b]

                @pl.when(c >= 2)
                def _drain_out():
                    out_desc(c - 2, b).wait()

                for d in in_descs(c, b):
                    d.wait()
                compute(krows, qc, outc)
                out_desc(c, b).start()

                @pl.when(c + 2 < base_ch)
                def _prefetch():
                    for d in in_descs(c + 2, b):
                        d.start()

        out_desc(base_ch - 2, 0).wait()
        out_desc(base_ch - 1, 1).wait()

        @pl.when(wid < rem)
        def _extra():
            g = NW * base_ch + wid
            nbase = g * CH_NODES
            ebase = nbase * K
            pltpu.sync_copy(idx_hbm.at[pl.ds(ebase, CH_EDGES)], idx_x)
            pltpu.async_copy(k_hbm.at[idx_x], krows0, in_sem0).wait()
            pltpu.sync_copy(qes_hbm.at[pl.ds(nbase, CH_NODES)], qc0)
            compute(krows0, qc0, outc0)
            pltpu.sync_copy(outc0, out_hbm.at[pl.ds(ebase, CH_EDGES)])

    return sc_kernel


def kernel(x_edge, node_irreps_input, f_sparse_idx_node, Wq, bq, Wk, bk,
           W1, b1, ln_g, ln_b, W2, b2):
    n = node_irreps_input.shape[0]
    xe2 = x_edge.reshape(n, K * 16)
    idx_flat = f_sparse_idx_node.astype(jnp.int32).reshape(n * K)
    eye = jnp.eye(K, dtype=jnp.float32)
    W1bd = jnp.kron(eye, W1)
    b1bd = jnp.tile(b1, K).reshape(1, K * 32)
    mred = jnp.kron(eye, jnp.full((32, 1), 1.0 / 32.0, jnp.float32))
    mbc = jnp.kron(eye, jnp.ones((1, 32), jnp.float32))
    lngbd = jnp.tile(ln_g, K).reshape(1, K * 32)
    lnbbd = jnp.tile(ln_b, K).reshape(1, K * 32)
    W2bd = jnp.kron(eye, W2)
    b2bd = jnp.tile(b2, K).reshape(1, K * H)
    qes, ktab = _tc_stage(
        node_irreps_input, xe2, Wq, bq.reshape(1, OUT), Wk, bk.reshape(1, OUT),
        W1bd, b1bd, mred, mbc, lngbd, lnbbd, W2bd, b2bd)
    alpha = _make_sc_kernel(n)(qes, ktab, idx_flat)
    return alpha.reshape(n, K, H)
